# baseline (speedup 1.0000x reference)
import functools

import numpy as np
import jax
import jax.numpy as jnp
from jax import lax
from jax.experimental import pallas as pl
from jax.experimental.pallas import tpu as pltpu
from jax.experimental.pallas import tpu_sc as plsc

_NUM_CLASSES = 80
_A = 8400
_G = 60
_KC = 10
_B = 8
_NW = 32
_ROWS_W = 40
_VREGS_W = 320
_OUT_W = _VREGS_W * 16
_G1_BASE = 20 * 5120
_G2_BASE = _G1_BASE + 8 * 3200
_OUT_TOTAL = _G2_BASE + 4 * 1600


def _sc_gather_tables():
    specs = [
        (20, 32, 160, 8),
        (8, 40, 80, 16),
        (4, 40, 40, 32),
    ]
    rows_by_scale = []
    for _, _, _, step in specs:
        ys = np.arange(step // 2, 640, step)
        rows_by_scale.append(
            (np.arange(_B)[:, None] * 640 + ys[None, :]).reshape(-1))
    row_ids = np.zeros((_NW, _ROWS_W), np.int64)
    lri = np.zeros((_NW, _OUT_W), np.int64)
    ci = np.zeros((_NW, _OUT_W), np.int64)
    t0 = 0
    for (nt, rpt, opr, step), rows in zip(specs, rows_by_scale):
        nout = rpt * opr
        j = np.arange(nout)
        r = j // opr
        k = j % opr
        col = (2 * step) * (k // 2) + step + (k % 2)
        for t in range(nt):
            row_ids[t0 + t, :rpt] = rows[t * rpt:(t + 1) * rpt]
            lri[t0 + t, :nout] = r
            ci[t0 + t, :nout] = col
        t0 += nt
    return (jnp.asarray(row_ids.reshape(-1), jnp.int32),
            jnp.asarray(lri.reshape(-1), jnp.int32),
            jnp.asarray(ci.reshape(-1), jnp.int32))


def _sc_mask_gather(masks):
    table = masks.reshape(_B * 640, 1280)
    rid, lri, ci = _sc_gather_tables()
    mesh = plsc.VectorSubcoreMesh(core_axis_name="c", subcore_axis_name="s")

    @functools.partial(
        pl.kernel, mesh=mesh,
        compiler_params=pltpu.CompilerParams(needs_layout_passes=False),
        out_type=(jax.ShapeDtypeStruct((_G1_BASE,), jnp.float32),
                  jax.ShapeDtypeStruct((_G2_BASE - _G1_BASE,), jnp.float32),
                  jax.ShapeDtypeStruct((_OUT_TOTAL - _G2_BASE,), jnp.float32)),
        scratch_types=[
            pltpu.VMEM((_ROWS_W,), jnp.int32),
            pltpu.VMEM((_ROWS_W, 1280), jnp.float32),
            pltpu.VMEM((_OUT_W,), jnp.int32),
            pltpu.VMEM((_OUT_W,), jnp.int32),
            pltpu.VMEM((_OUT_W,), jnp.float32),
            pltpu.SemaphoreType.DMA,
        ],
    )
    def gather_k(table_hbm, rid_hbm, lri_hbm, ci_hbm, out0_hbm, out1_hbm,
                 out2_hbm, rid_v, rows_v, lri_v, ci_v, out_v, sem):
        wid = lax.axis_index("s") * 2 + lax.axis_index("c")
        pltpu.sync_copy(rid_hbm.at[pl.ds(wid * _ROWS_W, _ROWS_W)], rid_v)
        pltpu.sync_copy(lri_hbm.at[pl.ds(wid * _OUT_W, _OUT_W)], lri_v)
        pltpu.sync_copy(ci_hbm.at[pl.ds(wid * _OUT_W, _OUT_W)], ci_v)
        pltpu.async_copy(table_hbm.at[rid_v], rows_v, sem).wait()

        @pl.loop(0, _VREGS_W)
        def _(j):
            rr = lri_v[pl.ds(j * 16, 16)]
            cc = ci_v[pl.ds(j * 16, 16)]
            out_v[pl.ds(j * 16, 16)] = plsc.load_gather(rows_v, [rr, cc])

        @pl.when(wid < 20)
        def _():
            pltpu.sync_copy(out_v, out0_hbm.at[pl.ds(wid * 5120, 5120)])

        @pl.when((wid >= 20) & (wid < 28))
        def _():
            pltpu.sync_copy(out_v.at[pl.ds(0, 3200)],
                            out1_hbm.at[pl.ds((wid - 20) * 3200, 3200)])

        @pl.when(wid >= 28)
        def _():
            pltpu.sync_copy(out_v.at[pl.ds(0, 1600)],
                            out2_hbm.at[pl.ds((wid - 28) * 1600, 1600)])

    b0, b1, b2 = gather_k(table, rid, lri, ci)
    return (b0.reshape(_B, 6400, 2), b1.reshape(_B, 1600, 2),
            b2.reshape(_B, 400, 2))


def _softplus(x):
    return jnp.maximum(x, 0.0) + jnp.log1p(jnp.exp(-jnp.abs(x)))


def _mask_loss_kernel(mr0_ref, mr1_ref, mr2_ref, o4c_ref, out_ref):
    f32 = jnp.float32
    t_mask = jnp.float32(0.0)
    num_m = jnp.float32(0.0)
    o4c = o4c_ref[0]
    for ref, lo, hi in ((mr0_ref, 0, 6400), (mr1_ref, 6400, 8000),
                        (mr2_ref, 8000, 8400)):
        x = ref[0]
        bm = x[:, 0:1]
        tm = x[:, 1:2]
        idx = ((bm + tm) > 0.0).astype(f32)
        x4 = o4c[lo:hi, :]
        t_mask = t_mask + jnp.sum((_softplus(x4) - x4 * tm) * idx)
        num_m = num_m + jnp.sum(idx)
    lane_o = jax.lax.broadcasted_iota(jnp.int32, (1, 1, 128), 2).astype(f32)
    out_ref[...] = (jnp.where(lane_o == 0.0, t_mask, 0.0)
                    + jnp.where(lane_o == 1.0, num_m, 0.0))


def _loss_kernel(tgt_ref, out_ref_n, o6T_ref, gT_ref, sT_ref, out_ref):
    f32 = jnp.float32
    tgt = tgt_ref[0]
    tcls = tgt[:, 0:1]
    gx = tgt[:, 1:2]
    gy = tgt[:, 2:3]
    gw = tgt[:, 3:4]
    gh = tgt[:, 4:5]
    validf = (tcls > 0.0).astype(f32)

    o6T = o6T_ref[0]
    px = o6T[0:1, :]
    py = o6T[1:2, :]
    pw = o6T[2:3, :]
    ph = o6T[3:4, :]

    sT = sT_ref[...]
    cx = (gT_ref[0:1, :] + 0.5) * sT
    cy = (gT_ref[1:2, :] + 0.5) * sT
    rad = 2.5 * sT

    gx1 = gx - gw * 0.5
    gx2 = gx + gw * 0.5
    gy1 = gy - gh * 0.5
    gy2 = gy + gh * 0.5
    px1 = px - pw * 0.5
    px2 = px + pw * 0.5
    py1 = py - ph * 0.5
    py2 = py + ph * 0.5
    iw = jnp.clip(jnp.minimum(gx2, px2) - jnp.maximum(gx1, px1), 0.0, None)
    ih = jnp.clip(jnp.minimum(gy2, py2) - jnp.maximum(gy1, py1), 0.0, None)
    inter = iw * ih
    area_g = gw * gh
    area_p = pw * ph
    iou = inter / (area_g + area_p - inter + 1e-8) * validf

    in_box = ((cx >= gx1) & (cx <= gx2) & (cy >= gy1) & (cy <= gy2))
    in_ctr = ((jnp.abs(cx - gx) < rad) & (jnp.abs(cy - gy) < rad))
    geomf = (in_box | in_ctr).astype(f32)
    cost = 1e5 * (1.0 - geomf) + 1e5 * (1.0 - validf) - iou
    iou_g = iou * geomf

    lane = jax.lax.broadcasted_iota(jnp.int32, (_G, _A), 1).astype(f32)
    work = cost
    s = jnp.zeros((_G, 1), f32)
    cv = []
    iv = []
    for _ in range(_KC):
        m = jnp.min(work, axis=1, keepdims=True)
        is_min = work == m
        first = jnp.min(jnp.where(is_min, lane, 1e9), axis=1,
                        keepdims=True)
        pick = is_min & (lane == first)
        s = s + jnp.sum(jnp.where(pick, iou_g, 0.0), axis=1, keepdims=True)
        cv.append(m)
        iv.append(first)
        work = jnp.where(pick, 1e30, work)

    kf = jnp.clip(jnp.floor(s), 1.0, float(_KC))
    thr = jnp.zeros((_G, 1), f32)
    for j in range(_KC):
        thr = thr + jnp.where(kf == float(j + 1), cv[j], 0.0)
    tie_j = jnp.full((_G, 1), -1.0, f32)
    for j in range(_KC):
        sel = (float(j) < kf) & (cv[j] == thr)
        tie_j = jnp.maximum(tie_j, jnp.where(sel, iv[j], -1.0))

    matching = (validf * ((cost < thr)
                          | ((cost == thr) & (lane <= tie_j))).astype(f32))
    colsum = jnp.sum(matching, axis=0, keepdims=True)
    conflict = colsum > 1.0
    mc = jnp.min(cost, axis=0, keepdims=True)
    gidx = jax.lax.broadcasted_iota(jnp.int32, (_G, _A), 0).astype(f32)
    firstg = jnp.min(jnp.where(cost == mc, gidx, 1e9), axis=0, keepdims=True)
    is_amin = (gidx == firstg).astype(f32)
    keep = matching * jnp.where(conflict, is_amin, 1.0)

    fg = jnp.sum(keep, axis=0, keepdims=True)
    num_fg = jnp.sum(fg)
    num_gt = jnp.sum(validf)
    pred_ious = jnp.sum(keep * iou, axis=0, keepdims=True)
    mgx = jnp.sum(keep * gx, axis=0, keepdims=True)
    mgy = jnp.sum(keep * gy, axis=0, keepdims=True)
    mgw = jnp.sum(keep * gw, axis=0, keepdims=True)
    mgh = jnp.sum(keep * gh, axis=0, keepdims=True)

    tlx = jnp.maximum(px1, mgx - mgw * 0.5)
    brx = jnp.minimum(px2, mgx + mgw * 0.5)
    tly = jnp.maximum(py1, mgy - mgh * 0.5)
    bry = jnp.minimum(py2, mgy + mgh * 0.5)
    rw = jnp.clip(brx - tlx, 0.0, None)
    rh = jnp.clip(bry - tly, 0.0, None)
    rinter = rw * rh
    runion = pw * ph + mgw * mgh - rinter + 1e-16
    riou = rinter / runion
    t_reg = jnp.sum((1.0 - riou * riou) * fg)

    x5 = o6T[5:6, :]
    t_iou = jnp.sum((_softplus(x5) - x5 * pred_ious) * fg)

    xc = out_ref_n[0][:, 6:]
    sp_sum = jnp.sum(_softplus(xc))
    m_gc = jax.lax.dot_general(keep, xc, (((1,), (0,)), ((), ())),
                               preferred_element_type=f32)
    ccol = jax.lax.broadcasted_iota(jnp.int32, (_G, _NUM_CLASSES), 1).astype(f32)
    cg = jnp.clip(tcls - 1.0, 0.0, float(_NUM_CLASSES - 1))
    corr = jnp.sum(m_gc * (ccol == cg).astype(f32))
    t_cls = sp_sum - corr

    lane_o = jax.lax.broadcasted_iota(jnp.int32, (1, 1, 128), 2).astype(f32)
    acc = (jnp.where(lane_o == 0.0, t_iou, 0.0)
           + jnp.where(lane_o == 1.0, t_reg, 0.0)
           + jnp.where(lane_o == 2.0, t_cls, 0.0)
           + jnp.where(lane_o == 4.0, num_fg, 0.0)
           + jnp.where(lane_o == 5.0, num_gt, 0.0))
    out_ref[...] = acc


def kernel(targets, strides, grids, outputs, regs, masks, use_augs):
    B = outputs.shape[0]
    o6T = jnp.swapaxes(outputs[:, :, :6], 1, 2)
    mr0, mr1, mr2 = _sc_mask_gather(masks)
    o4c = outputs[:, :, 4:5]
    gT = grids.T
    sT = strides.T

    partials = pl.pallas_call(
        _loss_kernel,
        grid=(B,),
        in_specs=[
            pl.BlockSpec((1, _G, 5), lambda i: (i, 0, 0)),
            pl.BlockSpec((1, _A, 6 + _NUM_CLASSES), lambda i: (i, 0, 0)),
            pl.BlockSpec((1, 6, _A), lambda i: (i, 0, 0)),
            pl.BlockSpec((2, _A), lambda i: (0, 0)),
            pl.BlockSpec((1, _A), lambda i: (0, 0)),
        ],
        out_specs=pl.BlockSpec((1, 1, 128), lambda i: (i, 0, 0)),
        out_shape=jax.ShapeDtypeStruct((B, 1, 128), jnp.float32),
    )(targets, outputs, o6T, gT, sT)

    mask_partials = pl.pallas_call(
        _mask_loss_kernel,
        grid=(B,),
        in_specs=[
            pl.BlockSpec((1, 6400, 2), lambda i: (i, 0, 0)),
            pl.BlockSpec((1, 1600, 2), lambda i: (i, 0, 0)),
            pl.BlockSpec((1, 400, 2), lambda i: (i, 0, 0)),
            pl.BlockSpec((1, _A, 1), lambda i: (i, 0, 0)),
        ],
        out_specs=pl.BlockSpec((1, 1, 128), lambda i: (i, 0, 0)),
        out_shape=jax.ShapeDtypeStruct((B, 1, 128), jnp.float32),
    )(mr0, mr1, mr2, o4c)

    sums = jnp.sum(partials[:, 0, :6], axis=0)
    msums = jnp.sum(mask_partials[:, 0, :2], axis=0)
    t_mask, num_m = msums[0], msums[1]
    t_iou, t_reg, t_cls, num_f, num_g = [sums[i] for i in (0, 1, 2, 4, 5)]
    num_f = jnp.maximum(num_f, 1.0)
    num_g = jnp.maximum(num_g, 1.0)
    num_m = jnp.maximum(num_m, 1.0)
    iou_loss = t_iou / num_f
    reg_loss = t_reg / num_f * 5.0
    cls_loss = t_cls / num_f
    mask_loss = t_mask / num_m * 2.0
    total = iou_loss + reg_loss + cls_loss + mask_loss
    return (total, iou_loss, reg_loss, cls_loss, mask_loss, num_f / num_g)


# speedup vs baseline: 1.3125x; 1.3125x over previous
import functools

import numpy as np
import jax
import jax.numpy as jnp
from jax import lax
from jax.experimental import pallas as pl
from jax.experimental.pallas import tpu as pltpu
from jax.experimental.pallas import tpu_sc as plsc

_NUM_CLASSES = 80
_A = 8400
_G = 60
_KC = 10
_B = 8
_NW = 32
_ROWS_W = 40
_PX_W = 2560
_VREGS_W = _PX_W // 16


def _sc_gather_tables():
    specs = [
        (20, 32, 80, 8),
        (8, 40, 40, 16),
        (4, 40, 20, 32),
    ]
    row_ids = np.zeros((_NW, _ROWS_W), np.int64)
    lri = np.zeros((_NW, _PX_W), np.int64)
    ci = np.zeros((_NW, _PX_W), np.int64)
    t0 = 0
    for nt, rpt, ppr, step in specs:
        ys = np.arange(step // 2, 640, step)
        rows = (np.arange(_B)[:, None] * 640 + ys[None, :]).reshape(-1)
        npx = rpt * ppr
        j = np.arange(npx)
        r = j // ppr
        col0 = (2 * step) * (j % ppr) + step
        for t in range(nt):
            row_ids[t0 + t, :rpt] = rows[t * rpt:(t + 1) * rpt]
            lri[t0 + t, :npx] = r
            ci[t0 + t, :npx] = col0
        t0 += nt
    return (jnp.asarray(row_ids.reshape(-1), jnp.int32),
            jnp.asarray(lri.reshape(-1), jnp.int32),
            jnp.asarray(ci.reshape(-1), jnp.int32))


def _sc_mask_gather(masks):
    table = masks.reshape(_B * 640, 1280)
    rid, lri, ci = _sc_gather_tables()
    mesh = plsc.VectorSubcoreMesh(core_axis_name="c", subcore_axis_name="s")

    @functools.partial(
        pl.kernel, mesh=mesh,
        compiler_params=pltpu.CompilerParams(needs_layout_passes=False),
        out_type=tuple(jax.ShapeDtypeStruct((_B * n,), jnp.float32)
                       for n in (6400, 6400, 1600, 1600, 400, 400)),
        scratch_types=[
            pltpu.VMEM((_ROWS_W,), jnp.int32),
            pltpu.VMEM((_ROWS_W, 1280), jnp.float32),
            pltpu.VMEM((_PX_W,), jnp.int32),
            pltpu.VMEM((_PX_W,), jnp.int32),
            pltpu.VMEM((_PX_W,), jnp.float32),
            pltpu.VMEM((_PX_W,), jnp.float32),
            pltpu.SemaphoreType.DMA,
        ],
    )
    def gather_k(table_hbm, rid_hbm, lri_hbm, ci_hbm,
                 bm0_hbm, tm0_hbm, bm1_hbm, tm1_hbm, bm2_hbm, tm2_hbm,
                 rid_v, rows_v, lri_v, ci_v, bm_v, tm_v, sem):
        wid = lax.axis_index("s") * 2 + lax.axis_index("c")
        pltpu.sync_copy(rid_hbm.at[pl.ds(wid * _ROWS_W, _ROWS_W)], rid_v)
        pltpu.sync_copy(lri_hbm.at[pl.ds(wid * _PX_W, _PX_W)], lri_v)
        pltpu.sync_copy(ci_hbm.at[pl.ds(wid * _PX_W, _PX_W)], ci_v)
        pltpu.async_copy(table_hbm.at[rid_v], rows_v, sem).wait()

        @pl.loop(0, _VREGS_W)
        def _(j):
            rr = lri_v[pl.ds(j * 16, 16)]
            cc = ci_v[pl.ds(j * 16, 16)]
            bm_v[pl.ds(j * 16, 16)] = plsc.load_gather(rows_v, [rr, cc])
            tm_v[pl.ds(j * 16, 16)] = plsc.load_gather(rows_v, [rr, cc + 1])

        @pl.when(wid < 20)
        def _():
            pltpu.sync_copy(bm_v, bm0_hbm.at[pl.ds(wid * 2560, 2560)])
            pltpu.sync_copy(tm_v, tm0_hbm.at[pl.ds(wid * 2560, 2560)])

        @pl.when((wid >= 20) & (wid < 28))
        def _():
            pltpu.sync_copy(bm_v.at[pl.ds(0, 1600)],
                            bm1_hbm.at[pl.ds((wid - 20) * 1600, 1600)])
            pltpu.sync_copy(tm_v.at[pl.ds(0, 1600)],
                            tm1_hbm.at[pl.ds((wid - 20) * 1600, 1600)])

        @pl.when(wid >= 28)
        def _():
            pltpu.sync_copy(bm_v.at[pl.ds(0, 800)],
                            bm2_hbm.at[pl.ds((wid - 28) * 800, 800)])
            pltpu.sync_copy(tm_v.at[pl.ds(0, 800)],
                            tm2_hbm.at[pl.ds((wid - 28) * 800, 800)])

    outs = gather_k(table, rid, lri, ci)
    ns = (6400, 6400, 1600, 1600, 400, 400)
    return tuple(o.reshape(_B, 1, n) for o, n in zip(outs, ns))


def _softplus(x):
    return jnp.maximum(x, 0.0) + jnp.log1p(jnp.exp(-jnp.abs(x)))


def _loss_kernel(tgt_ref, out_ref_n, o6T_ref, gT_ref, sT_ref,
                 bm0_ref, tm0_ref, bm1_ref, tm1_ref, bm2_ref, tm2_ref,
                 out_ref):
    f32 = jnp.float32
    tgt = tgt_ref[0]
    tcls = tgt[:, 0:1]
    gx = tgt[:, 1:2]
    gy = tgt[:, 2:3]
    gw = tgt[:, 3:4]
    gh = tgt[:, 4:5]
    validf = (tcls > 0.0).astype(f32)

    o6T = o6T_ref[0]
    px = o6T[0:1, :]
    py = o6T[1:2, :]
    pw = o6T[2:3, :]
    ph = o6T[3:4, :]

    sT = sT_ref[...]
    cx = (gT_ref[0:1, :] + 0.5) * sT
    cy = (gT_ref[1:2, :] + 0.5) * sT
    rad = 2.5 * sT

    gx1 = gx - gw * 0.5
    gx2 = gx + gw * 0.5
    gy1 = gy - gh * 0.5
    gy2 = gy + gh * 0.5
    px1 = px - pw * 0.5
    px2 = px + pw * 0.5
    py1 = py - ph * 0.5
    py2 = py + ph * 0.5
    iw = jnp.clip(jnp.minimum(gx2, px2) - jnp.maximum(gx1, px1), 0.0, None)
    ih = jnp.clip(jnp.minimum(gy2, py2) - jnp.maximum(gy1, py1), 0.0, None)
    inter = iw * ih
    area_g = gw * gh
    area_p = pw * ph
    iou = inter / (area_g + area_p - inter + 1e-8) * validf

    in_box = ((cx >= gx1) & (cx <= gx2) & (cy >= gy1) & (cy <= gy2))
    in_ctr = ((jnp.abs(cx - gx) < rad) & (jnp.abs(cy - gy) < rad))
    geomf = (in_box | in_ctr).astype(f32)
    cost = 1e5 * (1.0 - geomf) + 1e5 * (1.0 - validf) - iou
    iou_g = iou * geomf

    lane = jax.lax.broadcasted_iota(jnp.int32, (_G, _A), 1).astype(f32)
    work = cost
    s = jnp.zeros((_G, 1), f32)
    cv = []
    iv = []
    for _ in range(_KC):
        m = jnp.min(work, axis=1, keepdims=True)
        is_min = work == m
        first = jnp.min(jnp.where(is_min, lane, 1e9), axis=1,
                        keepdims=True)
        pick = is_min & (lane == first)
        s = s + jnp.sum(jnp.where(pick, iou_g, 0.0), axis=1, keepdims=True)
        cv.append(m)
        iv.append(first)
        work = jnp.where(pick, 1e30, work)

    kf = jnp.clip(jnp.floor(s), 1.0, float(_KC))
    thr = jnp.zeros((_G, 1), f32)
    for j in range(_KC):
        thr = thr + jnp.where(kf == float(j + 1), cv[j], 0.0)
    tie_j = jnp.full((_G, 1), -1.0, f32)
    for j in range(_KC):
        sel = (float(j) < kf) & (cv[j] == thr)
        tie_j = jnp.maximum(tie_j, jnp.where(sel, iv[j], -1.0))

    matching = (validf * ((cost < thr)
                          | ((cost == thr) & (lane <= tie_j))).astype(f32))
    colsum = jnp.sum(matching, axis=0, keepdims=True)
    conflict = colsum > 1.0
    mc = jnp.min(cost, axis=0, keepdims=True)
    gidx = jax.lax.broadcasted_iota(jnp.int32, (_G, _A), 0).astype(f32)
    firstg = jnp.min(jnp.where(cost == mc, gidx, 1e9), axis=0, keepdims=True)
    is_amin = (gidx == firstg).astype(f32)
    keep = matching * jnp.where(conflict, is_amin, 1.0)

    fg = jnp.sum(keep, axis=0, keepdims=True)
    num_fg = jnp.sum(fg)
    num_gt = jnp.sum(validf)
    pred_ious = jnp.sum(keep * iou, axis=0, keepdims=True)
    mgx = jnp.sum(keep * gx, axis=0, keepdims=True)
    mgy = jnp.sum(keep * gy, axis=0, keepdims=True)
    mgw = jnp.sum(keep * gw, axis=0, keepdims=True)
    mgh = jnp.sum(keep * gh, axis=0, keepdims=True)

    tlx = jnp.maximum(px1, mgx - mgw * 0.5)
    brx = jnp.minimum(px2, mgx + mgw * 0.5)
    tly = jnp.maximum(py1, mgy - mgh * 0.5)
    bry = jnp.minimum(py2, mgy + mgh * 0.5)
    rw = jnp.clip(brx - tlx, 0.0, None)
    rh = jnp.clip(bry - tly, 0.0, None)
    rinter = rw * rh
    runion = pw * ph + mgw * mgh - rinter + 1e-16
    riou = rinter / runion
    t_reg = jnp.sum((1.0 - riou * riou) * fg)

    x5 = o6T[5:6, :]
    t_iou = jnp.sum((_softplus(x5) - x5 * pred_ious) * fg)

    xc = out_ref_n[0][:, 6:]
    sp_sum = jnp.sum(_softplus(xc))
    m_gc = jax.lax.dot_general(keep, xc, (((1,), (0,)), ((), ())),
                               preferred_element_type=f32)
    ccol = jax.lax.broadcasted_iota(jnp.int32, (_G, _NUM_CLASSES), 1).astype(f32)
    cg = jnp.clip(tcls - 1.0, 0.0, float(_NUM_CLASSES - 1))
    corr = jnp.sum(m_gc * (ccol == cg).astype(f32))
    t_cls = sp_sum - corr

    x4 = o6T[4:5, :]
    t_mask = jnp.float32(0.0)
    num_m = jnp.float32(0.0)
    for b_ref, t_ref, lo, hi in ((bm0_ref, tm0_ref, 0, 6400),
                                 (bm1_ref, tm1_ref, 6400, 8000),
                                 (bm2_ref, tm2_ref, 8000, 8400)):
        bmr = b_ref[0]
        tmr = t_ref[0]
        idx = ((bmr + tmr) > 0.0).astype(f32)
        x4s = x4[:, lo:hi]
        t_mask = t_mask + jnp.sum((_softplus(x4s) - x4s * tmr) * idx)
        num_m = num_m + jnp.sum(idx)

    lane_o = jax.lax.broadcasted_iota(jnp.int32, (1, 1, 128), 2).astype(f32)
    acc = (jnp.where(lane_o == 0.0, t_iou, 0.0)
           + jnp.where(lane_o == 1.0, t_reg, 0.0)
           + jnp.where(lane_o == 2.0, t_cls, 0.0)
           + jnp.where(lane_o == 3.0, t_mask, 0.0)
           + jnp.where(lane_o == 4.0, num_fg, 0.0)
           + jnp.where(lane_o == 5.0, num_gt, 0.0)
           + jnp.where(lane_o == 6.0, num_m, 0.0))
    out_ref[...] = acc


def kernel(targets, strides, grids, outputs, regs, masks, use_augs):
    B = outputs.shape[0]
    o6T = jnp.swapaxes(outputs[:, :, :6], 1, 2)
    bm0, tm0, bm1, tm1, bm2, tm2 = _sc_mask_gather(masks)
    gT = grids.T
    sT = strides.T

    partials = pl.pallas_call(
        _loss_kernel,
        grid=(B,),
        in_specs=[
            pl.BlockSpec((1, _G, 5), lambda i: (i, 0, 0)),
            pl.BlockSpec((1, _A, 6 + _NUM_CLASSES), lambda i: (i, 0, 0)),
            pl.BlockSpec((1, 6, _A), lambda i: (i, 0, 0)),
            pl.BlockSpec((2, _A), lambda i: (0, 0)),
            pl.BlockSpec((1, _A), lambda i: (0, 0)),
            pl.BlockSpec((1, 1, 6400), lambda i: (i, 0, 0)),
            pl.BlockSpec((1, 1, 6400), lambda i: (i, 0, 0)),
            pl.BlockSpec((1, 1, 1600), lambda i: (i, 0, 0)),
            pl.BlockSpec((1, 1, 1600), lambda i: (i, 0, 0)),
            pl.BlockSpec((1, 1, 400), lambda i: (i, 0, 0)),
            pl.BlockSpec((1, 1, 400), lambda i: (i, 0, 0)),
        ],
        out_specs=pl.BlockSpec((1, 1, 128), lambda i: (i, 0, 0)),
        out_shape=jax.ShapeDtypeStruct((B, 1, 128), jnp.float32),
    )(targets, outputs, o6T, gT, sT, bm0, tm0, bm1, tm1, bm2, tm2)

    sums = jnp.sum(partials[:, 0, :7], axis=0)
    t_iou, t_reg, t_cls, t_mask, num_f, num_g, num_m = [sums[i]
                                                        for i in range(7)]
    num_f = jnp.maximum(num_f, 1.0)
    num_g = jnp.maximum(num_g, 1.0)
    num_m = jnp.maximum(num_m, 1.0)
    iou_loss = t_iou / num_f
    reg_loss = t_reg / num_f * 5.0
    cls_loss = t_cls / num_f
    mask_loss = t_mask / num_m * 2.0
    total = iou_loss + reg_loss + cls_loss + mask_loss
    return (total, iou_loss, reg_loss, cls_loss, mask_loss, num_f / num_g)


# speedup vs baseline: 1.3631x; 1.0386x over previous
import functools

import numpy as np
import jax
import jax.numpy as jnp
from jax import lax
from jax.experimental import pallas as pl
from jax.experimental.pallas import tpu as pltpu
from jax.experimental.pallas import tpu_sc as plsc

_NUM_CLASSES = 80
_A = 8400
_G = 60
_KC = 10
_B = 8
_NW = 32
_ROWS_W = 40
_PX_W = 2560
_VREGS_W = _PX_W // 16


def _sc_gather_tables():
    specs = [
        (20, 32, 80, 8),
        (8, 40, 40, 16),
        (4, 40, 20, 32),
    ]
    row_ids = np.zeros((_NW, _ROWS_W), np.int64)
    lri = np.zeros((_NW, _PX_W), np.int64)
    ci = np.zeros((_NW, _PX_W), np.int64)
    t0 = 0
    for nt, rpt, ppr, step in specs:
        ys = np.arange(step // 2, 640, step)
        rows = (np.arange(_B)[:, None] * 640 + ys[None, :]).reshape(-1)
        npx = rpt * ppr
        j = np.arange(npx)
        r = j // ppr
        col0 = (2 * step) * (j % ppr) + step
        for t in range(nt):
            row_ids[t0 + t, :rpt] = rows[t * rpt:(t + 1) * rpt]
            lri[t0 + t, :npx] = r
            ci[t0 + t, :npx] = col0
        t0 += nt
    return (jnp.asarray(row_ids.reshape(-1), jnp.int32),
            jnp.asarray(lri.reshape(-1), jnp.int32),
            jnp.asarray(ci.reshape(-1), jnp.int32))


def _sc_mask_gather(masks):
    table = masks.reshape(_B * 640, 1280)
    rid, lri, ci = _sc_gather_tables()
    mesh = plsc.VectorSubcoreMesh(core_axis_name="c", subcore_axis_name="s")

    @functools.partial(
        pl.kernel, mesh=mesh,
        compiler_params=pltpu.CompilerParams(needs_layout_passes=False),
        out_type=tuple(jax.ShapeDtypeStruct((_B * n,), jnp.float32)
                       for n in (6400, 6400, 1600, 1600, 400, 400)),
        scratch_types=[
            pltpu.VMEM((_ROWS_W,), jnp.int32),
            pltpu.VMEM((_ROWS_W, 1280), jnp.float32),
            pltpu.VMEM((_PX_W,), jnp.int32),
            pltpu.VMEM((_PX_W,), jnp.int32),
            pltpu.VMEM((_PX_W,), jnp.float32),
            pltpu.VMEM((_PX_W,), jnp.float32),
            pltpu.SemaphoreType.DMA,
        ],
    )
    def gather_k(table_hbm, rid_hbm, lri_hbm, ci_hbm,
                 bm0_hbm, tm0_hbm, bm1_hbm, tm1_hbm, bm2_hbm, tm2_hbm,
                 rid_v, rows_v, lri_v, ci_v, bm_v, tm_v, sem):
        wid = lax.axis_index("s") * 2 + lax.axis_index("c")
        pltpu.sync_copy(rid_hbm.at[pl.ds(wid * _ROWS_W, _ROWS_W)], rid_v)
        pltpu.sync_copy(lri_hbm.at[pl.ds(wid * _PX_W, _PX_W)], lri_v)
        pltpu.sync_copy(ci_hbm.at[pl.ds(wid * _PX_W, _PX_W)], ci_v)
        pltpu.async_copy(table_hbm.at[rid_v], rows_v, sem).wait()

        @pl.loop(0, _VREGS_W)
        def _(j):
            rr = lri_v[pl.ds(j * 16, 16)]
            cc = ci_v[pl.ds(j * 16, 16)]
            bm_v[pl.ds(j * 16, 16)] = plsc.load_gather(rows_v, [rr, cc])
            tm_v[pl.ds(j * 16, 16)] = plsc.load_gather(rows_v, [rr, cc + 1])

        @pl.when(wid < 20)
        def _():
            pltpu.sync_copy(bm_v, bm0_hbm.at[pl.ds(wid * 2560, 2560)])
            pltpu.sync_copy(tm_v, tm0_hbm.at[pl.ds(wid * 2560, 2560)])

        @pl.when((wid >= 20) & (wid < 28))
        def _():
            pltpu.sync_copy(bm_v.at[pl.ds(0, 1600)],
                            bm1_hbm.at[pl.ds((wid - 20) * 1600, 1600)])
            pltpu.sync_copy(tm_v.at[pl.ds(0, 1600)],
                            tm1_hbm.at[pl.ds((wid - 20) * 1600, 1600)])

        @pl.when(wid >= 28)
        def _():
            pltpu.sync_copy(bm_v.at[pl.ds(0, 800)],
                            bm2_hbm.at[pl.ds((wid - 28) * 800, 800)])
            pltpu.sync_copy(tm_v.at[pl.ds(0, 800)],
                            tm2_hbm.at[pl.ds((wid - 28) * 800, 800)])

    outs = gather_k(table, rid, lri, ci)
    ns = (6400, 6400, 1600, 1600, 400, 400)
    return tuple(o.reshape(_B, 1, n) for o, n in zip(outs, ns))


def _softplus(x):
    return jnp.maximum(x, 0.0) + jnp.log1p(jnp.exp(-jnp.abs(x)))


def _loss_kernel(tgt_ref, out_ref_n, gT_ref, sT_ref,
                 bm0_ref, tm0_ref, bm1_ref, tm1_ref, bm2_ref, tm2_ref,
                 out_ref):
    f32 = jnp.float32
    tgt = tgt_ref[0]
    tcls = tgt[:, 0:1]
    gx = tgt[:, 1:2]
    gy = tgt[:, 2:3]
    gw = tgt[:, 3:4]
    gh = tgt[:, 4:5]
    validf = (tcls > 0.0).astype(f32)

    X = out_ref_n[0]
    e8 = (jax.lax.broadcasted_iota(jnp.int32, (8, 6 + _NUM_CLASSES), 0)
          == jax.lax.broadcasted_iota(jnp.int32, (8, 6 + _NUM_CLASSES), 1)
          ).astype(f32)
    o6T = jax.lax.dot_general(e8, X, (((1,), (1,)), ((), ())),
                              preferred_element_type=f32)
    px = o6T[0:1, :]
    py = o6T[1:2, :]
    pw = o6T[2:3, :]
    ph = o6T[3:4, :]

    sT = sT_ref[...]
    cx = (gT_ref[0:1, :] + 0.5) * sT
    cy = (gT_ref[1:2, :] + 0.5) * sT
    rad = 2.5 * sT

    gx1 = gx - gw * 0.5
    gx2 = gx + gw * 0.5
    gy1 = gy - gh * 0.5
    gy2 = gy + gh * 0.5
    px1 = px - pw * 0.5
    px2 = px + pw * 0.5
    py1 = py - ph * 0.5
    py2 = py + ph * 0.5
    iw = jnp.clip(jnp.minimum(gx2, px2) - jnp.maximum(gx1, px1), 0.0, None)
    ih = jnp.clip(jnp.minimum(gy2, py2) - jnp.maximum(gy1, py1), 0.0, None)
    inter = iw * ih
    area_g = gw * gh
    area_p = pw * ph
    iou = inter / (area_g + area_p - inter + 1e-8) * validf

    in_box = ((cx >= gx1) & (cx <= gx2) & (cy >= gy1) & (cy <= gy2))
    in_ctr = ((jnp.abs(cx - gx) < rad) & (jnp.abs(cy - gy) < rad))
    geomf = (in_box | in_ctr).astype(f32)
    cost = 1e5 * (1.0 - geomf) + 1e5 * (1.0 - validf) - iou
    iou_g = iou * geomf

    lane = jax.lax.broadcasted_iota(jnp.int32, (_G, _A), 1).astype(f32)
    work = cost
    s = jnp.zeros((_G, 1), f32)
    cv = []
    iv = []
    for _ in range(_KC):
        m = jnp.min(work, axis=1, keepdims=True)
        is_min = work == m
        first = jnp.min(jnp.where(is_min, lane, 1e9), axis=1,
                        keepdims=True)
        pick = is_min & (lane == first)
        s = s + jnp.sum(jnp.where(pick, iou_g, 0.0), axis=1, keepdims=True)
        cv.append(m)
        iv.append(first)
        work = jnp.where(pick, 1e30, work)

    kf = jnp.clip(jnp.floor(s), 1.0, float(_KC))
    thr = jnp.zeros((_G, 1), f32)
    for j in range(_KC):
        thr = thr + jnp.where(kf == float(j + 1), cv[j], 0.0)
    tie_j = jnp.full((_G, 1), -1.0, f32)
    for j in range(_KC):
        sel = (float(j) < kf) & (cv[j] == thr)
        tie_j = jnp.maximum(tie_j, jnp.where(sel, iv[j], -1.0))

    matching = (validf * ((cost < thr)
                          | ((cost == thr) & (lane <= tie_j))).astype(f32))
    colsum = jnp.sum(matching, axis=0, keepdims=True)
    conflict = colsum > 1.0
    mc = jnp.min(cost, axis=0, keepdims=True)
    gidx = jax.lax.broadcasted_iota(jnp.int32, (_G, _A), 0).astype(f32)
    firstg = jnp.min(jnp.where(cost == mc, gidx, 1e9), axis=0, keepdims=True)
    is_amin = (gidx == firstg).astype(f32)
    keep = matching * jnp.where(conflict, is_amin, 1.0)

    fg = jnp.sum(keep, axis=0, keepdims=True)
    num_fg = jnp.sum(fg)
    num_gt = jnp.sum(validf)
    pred_ious = jnp.sum(keep * iou, axis=0, keepdims=True)
    mgx = jnp.sum(keep * gx, axis=0, keepdims=True)
    mgy = jnp.sum(keep * gy, axis=0, keepdims=True)
    mgw = jnp.sum(keep * gw, axis=0, keepdims=True)
    mgh = jnp.sum(keep * gh, axis=0, keepdims=True)

    tlx = jnp.maximum(px1, mgx - mgw * 0.5)
    brx = jnp.minimum(px2, mgx + mgw * 0.5)
    tly = jnp.maximum(py1, mgy - mgh * 0.5)
    bry = jnp.minimum(py2, mgy + mgh * 0.5)
    rw = jnp.clip(brx - tlx, 0.0, None)
    rh = jnp.clip(bry - tly, 0.0, None)
    rinter = rw * rh
    runion = pw * ph + mgw * mgh - rinter + 1e-16
    riou = rinter / runion
    t_reg = jnp.sum((1.0 - riou * riou) * fg)

    x5 = o6T[5:6, :]
    t_iou = jnp.sum((_softplus(x5) - x5 * pred_ious) * fg)

    xc = out_ref_n[0][:, 6:]
    sp_sum = jnp.sum(_softplus(xc))
    m_gc = jax.lax.dot_general(keep, xc, (((1,), (0,)), ((), ())),
                               preferred_element_type=f32)
    ccol = jax.lax.broadcasted_iota(jnp.int32, (_G, _NUM_CLASSES), 1).astype(f32)
    cg = jnp.clip(tcls - 1.0, 0.0, float(_NUM_CLASSES - 1))
    corr = jnp.sum(m_gc * (ccol == cg).astype(f32))
    t_cls = sp_sum - corr

    x4 = o6T[4:5, :]
    t_mask = jnp.float32(0.0)
    num_m = jnp.float32(0.0)
    for b_ref, t_ref, lo, hi in ((bm0_ref, tm0_ref, 0, 6400),
                                 (bm1_ref, tm1_ref, 6400, 8000),
                                 (bm2_ref, tm2_ref, 8000, 8400)):
        bmr = b_ref[0]
        tmr = t_ref[0]
        idx = ((bmr + tmr) > 0.0).astype(f32)
        x4s = x4[:, lo:hi]
        t_mask = t_mask + jnp.sum((_softplus(x4s) - x4s * tmr) * idx)
        num_m = num_m + jnp.sum(idx)

    lane_o = jax.lax.broadcasted_iota(jnp.int32, (1, 1, 128), 2).astype(f32)
    acc = (jnp.where(lane_o == 0.0, t_iou, 0.0)
           + jnp.where(lane_o == 1.0, t_reg, 0.0)
           + jnp.where(lane_o == 2.0, t_cls, 0.0)
           + jnp.where(lane_o == 3.0, t_mask, 0.0)
           + jnp.where(lane_o == 4.0, num_fg, 0.0)
           + jnp.where(lane_o == 5.0, num_gt, 0.0)
           + jnp.where(lane_o == 6.0, num_m, 0.0))
    out_ref[...] = acc


def kernel(targets, strides, grids, outputs, regs, masks, use_augs):
    B = outputs.shape[0]
    bm0, tm0, bm1, tm1, bm2, tm2 = _sc_mask_gather(masks)
    gT = grids.T
    sT = strides.T

    partials = pl.pallas_call(
        _loss_kernel,
        grid=(B,),
        in_specs=[
            pl.BlockSpec((1, _G, 5), lambda i: (i, 0, 0)),
            pl.BlockSpec((1, _A, 6 + _NUM_CLASSES), lambda i: (i, 0, 0)),
            pl.BlockSpec((2, _A), lambda i: (0, 0)),
            pl.BlockSpec((1, _A), lambda i: (0, 0)),
            pl.BlockSpec((1, 1, 6400), lambda i: (i, 0, 0)),
            pl.BlockSpec((1, 1, 6400), lambda i: (i, 0, 0)),
            pl.BlockSpec((1, 1, 1600), lambda i: (i, 0, 0)),
            pl.BlockSpec((1, 1, 1600), lambda i: (i, 0, 0)),
            pl.BlockSpec((1, 1, 400), lambda i: (i, 0, 0)),
            pl.BlockSpec((1, 1, 400), lambda i: (i, 0, 0)),
        ],
        out_specs=pl.BlockSpec((1, 1, 128), lambda i: (i, 0, 0)),
        out_shape=jax.ShapeDtypeStruct((B, 1, 128), jnp.float32),
    )(targets, outputs, gT, sT, bm0, tm0, bm1, tm1, bm2, tm2)

    sums = jnp.sum(partials[:, 0, :7], axis=0)
    t_iou, t_reg, t_cls, t_mask, num_f, num_g, num_m = [sums[i]
                                                        for i in range(7)]
    num_f = jnp.maximum(num_f, 1.0)
    num_g = jnp.maximum(num_g, 1.0)
    num_m = jnp.maximum(num_m, 1.0)
    iou_loss = t_iou / num_f
    reg_loss = t_reg / num_f * 5.0
    cls_loss = t_cls / num_f
    mask_loss = t_mask / num_m * 2.0
    total = iou_loss + reg_loss + cls_loss + mask_loss
    return (total, iou_loss, reg_loss, cls_loss, mask_loss, num_f / num_g)
